# baseline (device time: 91866 ns/iter reference)
import jax
import jax.numpy as jnp
from jax import lax
from jax.experimental import pallas as pl
from jax.experimental.pallas import tpu as pltpu

N_DEV = 4
B = 2
SQ = 256
SKV = 256
HQ = 4
DH = 64
BLK = 64
NBLK = SQ // BLK


def kernel(x, Wq, K_ext, V_ext, Wo):
    d_model = x.shape[-1]
    d_q = Wq.shape[-1]

    def body(x_ref, wq_ref, k_ref, v_ref, wo_ref, out_ref,
             kg_ref, vg_ref, ksend, krecv, vsend, vrecv):
        me = lax.axis_index("i")
        left = (me + N_DEV - 1) % N_DEV
        right = (me + 1) % N_DEV

        barrier = pltpu.get_barrier_semaphore()
        for nbr in (left, right):
            pl.semaphore_signal(barrier, inc=1, device_id=(nbr,),
                                device_id_type=pl.DeviceIdType.MESH)
        pl.semaphore_wait(barrier, 2)

        kg_ref[0] = k_ref[...]
        vg_ref[0] = v_ref[...]

        for h in range(N_DEV - 1):
            k_rdma = pltpu.make_async_remote_copy(
                src_ref=kg_ref.at[h], dst_ref=kg_ref.at[h + 1],
                send_sem=ksend.at[h], recv_sem=krecv.at[h],
                device_id=(right,), device_id_type=pl.DeviceIdType.MESH)
            v_rdma = pltpu.make_async_remote_copy(
                src_ref=vg_ref.at[h], dst_ref=vg_ref.at[h + 1],
                send_sem=vsend.at[h], recv_sem=vrecv.at[h],
                device_id=(right,), device_id_type=pl.DeviceIdType.MESH)
            k_rdma.start()
            v_rdma.start()
            k_rdma.wait()
            v_rdma.wait()

        for b in range(B):
            q_b = lax.dot_general(
                x_ref[b], wq_ref[...],
                (((1,), (0,)), ((), ())),
                preferred_element_type=jnp.float32)
            for qb in range(NBLK):
                rows = pl.ds(qb * BLK, BLK)
                ctx_heads = []
                for hh in range(HQ):
                    q_blk = q_b[qb * BLK:(qb + 1) * BLK,
                                hh * DH:(hh + 1) * DH]
                    ks = jnp.concatenate(
                        [kg_ref[d, b, rows, hh, :] for d in range(N_DEV)],
                        axis=0)
                    vs = jnp.concatenate(
                        [vg_ref[d, b, rows, hh, :] for d in range(N_DEV)],
                        axis=0)
                    s = lax.dot_general(
                        q_blk, ks, (((1,), (1,)), ((), ())),
                        preferred_element_type=jnp.float32) * 0.125
                    m = jnp.max(s, axis=1, keepdims=True)
                    e = jnp.exp(s - m)
                    w = e / jnp.sum(e, axis=1, keepdims=True)
                    ctx_heads.append(lax.dot_general(
                        w, vs, (((1,), (0,)), ((), ())),
                        preferred_element_type=jnp.float32))
                ctx = jnp.concatenate(ctx_heads, axis=1)
                out_ref[b, rows, :] = lax.dot_general(
                    ctx, wo_ref[...], (((1,), (0,)), ((), ())),
                    preferred_element_type=jnp.float32)

    return pl.pallas_call(
        body,
        out_shape=jax.ShapeDtypeStruct((B, SQ, d_model), jnp.float32),
        in_specs=[pl.BlockSpec(memory_space=pltpu.VMEM)] * 5,
        out_specs=pl.BlockSpec(memory_space=pltpu.VMEM),
        scratch_shapes=[
            pltpu.VMEM((N_DEV, B, SKV, HQ, DH), jnp.float32),
            pltpu.VMEM((N_DEV, B, SKV, HQ, DH), jnp.float32),
            pltpu.SemaphoreType.DMA((N_DEV - 1,)),
            pltpu.SemaphoreType.DMA((N_DEV - 1,)),
            pltpu.SemaphoreType.DMA((N_DEV - 1,)),
            pltpu.SemaphoreType.DMA((N_DEV - 1,)),
        ],
        compiler_params=pltpu.CompilerParams(collective_id=0),
    )(x, Wq, K_ext, V_ext, Wo)


# device time: 19440 ns/iter; 4.7256x vs baseline; 4.7256x over previous
import jax
import jax.numpy as jnp
from jax import lax
from jax.experimental import pallas as pl
from jax.experimental.pallas import tpu as pltpu

N_DEV = 4
B = 2
SQ = 256
SKV = 256
HQ = 4
DH = 64
F = HQ * DH
BLK = 64
BF = jnp.bfloat16


def kernel(x, Wq, K_ext, V_ext, Wo):
    d_model = x.shape[-1]
    K2 = K_ext.reshape(B, SKV, F).astype(BF)
    V2 = V_ext.reshape(B, SKV, F).astype(BF)

    def body(x_ref, wq_ref, k_ref, v_ref, wo_ref, out_ref, kvg_ref,
             ssem, rsem):
        me = lax.axis_index("i")
        left = (me + N_DEV - 1) % N_DEV
        right = (me + 1) % N_DEV

        barrier = pltpu.get_barrier_semaphore()
        for nbr in (left, right):
            pl.semaphore_signal(barrier, inc=1, device_id=(nbr,),
                                device_id_type=pl.DeviceIdType.MESH)
        pl.semaphore_wait(barrier, 2)

        def rc(src, dst, si, ri, dev):
            return pltpu.make_async_remote_copy(
                src_ref=src, dst_ref=dst, send_sem=ssem.at[si],
                recv_sem=rsem.at[ri], device_id=(dev,),
                device_id_type=pl.DeviceIdType.MESH)

        h1l_k = [rc(k_ref.at[b], kvg_ref.at[0, 2, b], b, b, left)
                 for b in range(B)]
        h1r_k = [rc(k_ref.at[b], kvg_ref.at[0, 1, b], 2 + b, 2 + b, right)
                 for b in range(B)]
        h1r_v = [rc(v_ref.at[b], kvg_ref.at[1, 1, b], 4 + b, 4 + b, right)
                 for b in range(B)]
        h1l_v = rc(v_ref, kvg_ref.at[1, 2], 6, 6, left)
        h2l = [rc(kvg_ref.at[0, 2, b], kvg_ref.at[0, 3, b], 7 + b, 7 + b,
                  left) for b in range(B)]
        h2r = [rc(kvg_ref.at[1, 1, b], kvg_ref.at[1, 3, b], 9 + b, 9 + b,
                  right) for b in range(B)]

        for d in h1l_k + h1r_k + h1r_v:
            d.start()

        rowblk = lax.broadcasted_iota(jnp.int32, (SQ, 2 * SKV), 0) // BLK
        colblk = lax.broadcasted_iota(jnp.int32, (SQ, 2 * SKV), 1) // BLK % 4
        mask = (colblk == rowblk).astype(jnp.float32)

        wq_bf = wq_ref[...].astype(BF)
        wo_bf = wo_ref[...].astype(BF)
        q = [(lax.dot_general(x_ref[b].astype(BF), wq_bf,
                              (((1,), (0,)), ((), ())),
                              preferred_element_type=jnp.float32)
              * 0.125).astype(BF)
             for b in range(B)]

        h1l_k[0].wait_recv()
        h2l[0].start()
        h1l_k[1].wait_recv()
        h2l[1].start()
        h1l_v.start()

        kvg_ref[0, 0] = k_ref[...]
        kvg_ref[1, 0] = v_ref[...]

        def scores(b, hh, lo):
            kk = kvg_ref[0, lo:lo + 2, b, :, pl.ds(hh * DH, DH)]
            kk = kk.reshape(2 * SKV, DH)
            q_h = q[b][:, hh * DH:(hh + 1) * DH]
            s = lax.dot_general(q_h, kk, (((1,), (1,)), ((), ())),
                                preferred_element_type=jnp.float32)
            e = jnp.exp(s) * mask
            return e.astype(BF), jnp.sum(e, axis=1, keepdims=True)

        def ctxdot(e, b, hh, lo):
            vv = kvg_ref[1, lo:lo + 2, b, :, pl.ds(hh * DH, DH)]
            vv = vv.reshape(2 * SKV, DH)
            return lax.dot_general(e, vv, (((1,), (0,)), ((), ())),
                                   preferred_element_type=jnp.float32)

        h1r_k[0].wait_recv()
        h1r_k[1].wait_recv()
        st = {}
        for hh in range(HQ):
            st[0, hh] = list(scores(0, hh, 0))
        h1r_v[0].wait_recv()
        h2r[0].start()
        for hh in range(HQ):
            st[1, hh] = list(scores(1, hh, 0))
        h1r_v[1].wait_recv()
        h2r[1].start()
        for b in range(B):
            for hh in range(HQ):
                st[b, hh].append(ctxdot(st[b, hh][0], b, hh, 0))

        h2l[0].wait_recv()
        h2l[1].wait_recv()
        for b in range(B):
            for hh in range(HQ):
                rec = st[b, hh]
                eB, dB = scores(b, hh, 2)
                rec.append(eB)
                rec[1] = 1.0 / (rec[1] + dB)

        h1l_v.wait_recv()
        h2r[0].wait_recv()
        h2r[1].wait_recv()
        for b in range(B):
            acc = None
            for hh in range(HQ):
                eA, r, ctxA, eB = st[b, hh]
                ctx = ((ctxA + ctxdot(eB, b, hh, 2)) * r).astype(BF)
                o = lax.dot_general(
                    ctx, wo_bf[hh * DH:(hh + 1) * DH, :],
                    (((1,), (0,)), ((), ())),
                    preferred_element_type=jnp.float32)
                acc = o if acc is None else acc + o
            out_ref[b] = acc

        for d in h1l_k + h1r_k + h1r_v + [h1l_v] + h2l + h2r:
            d.wait_send()

    return pl.pallas_call(
        body,
        out_shape=jax.ShapeDtypeStruct((B, SQ, d_model), jnp.float32),
        in_specs=[pl.BlockSpec(memory_space=pltpu.VMEM)] * 5,
        out_specs=pl.BlockSpec(memory_space=pltpu.VMEM),
        scratch_shapes=[
            pltpu.VMEM((2, N_DEV, B, SKV, F), BF),
            pltpu.SemaphoreType.DMA((11,)),
            pltpu.SemaphoreType.DMA((11,)),
        ],
        compiler_params=pltpu.CompilerParams(collective_id=0),
    )(x, Wq, K2, V2, Wo)
